# baseline (speedup 1.0000x reference)
import functools

import jax
import jax.numpy as jnp
from jax import lax
from jax.experimental import pallas as pl
from jax.experimental.pallas import tpu as pltpu
from jax.experimental.pallas import tpu_sc as plsc

_PAD = 1024
_NREAL = 1002
_NSAMP = 1000.0
_BIG = 1e30


def _prep_body(vc_ref, vcT_ref, qpad_ref, scal_ref, thr_ref, tab_ref):
    vrow = vc_ref[0]
    vcol = vcT_ref[0]
    qpad = qpad_ref[...]

    io_row = lax.broadcasted_iota(jnp.int32, (1, _PAD), 1).astype(jnp.float32)
    io_col = lax.broadcasted_iota(jnp.int32, (_PAD, 1), 0).astype(jnp.float32)

    before = (vrow < vcol) | ((vrow == vcol) & (io_row < io_col))
    before_f = jnp.where(before, 1.0, 0.0)
    ones_col = jnp.ones((_PAD, 1), jnp.float32)
    rank_col = jax.lax.dot_general(
        before_f, ones_col, (((1,), (0,)), ((), ())),
        preferred_element_type=jnp.float32)

    sn = jnp.sum(jnp.where(vrow < 0, 1.0, 0.0))
    sp = jnp.float32(_NREAL) - sn
    wn = (1.0 / sn) * _NSAMP
    wp = (1.0 / sp) * _NSAMP
    tot = sn * wn + sp * wp

    cw_col = jnp.where(io_col < sn, (io_col + 1.0) * wn,
                       sn * wn + (io_col + 1.0 - sn) * wp)
    tw = qpad * tot

    valid = io_col < jnp.float32(_NREAL)
    above = jnp.sum(jnp.where((cw_col <= tw) & valid, 1.0, 0.0),
                    axis=0, keepdims=True)
    below = jnp.clip(above - 1.0, 0.0, _NSAMP - 1.0)
    above = jnp.clip(above, 0.0, _NSAMP - 1.0)

    def sorted_at(r):
        return jnp.sum(jnp.where(rank_col == r, vcol, 0.0), axis=0, keepdims=True)

    def cw_at(r):
        return jnp.where(r < sn, (r + 1.0) * wn, sn * wn + (r + 1.0 - sn) * wp)

    vb, va = sorted_at(below), sorted_at(above)
    wb, wa = cw_at(below), cw_at(above)
    frac = (tw - wb) / (wa - wb + 1e-6)
    qv = vb + (va - vb) * frac

    w_in = scal_ref[0, 0, 0]
    b_in = scal_ref[0, 0, 1]
    m_in = scal_ref[0, 0, 2]
    v_in = scal_ref[0, 0, 3]
    w_o = scal_ref[0, 0, 4]
    b_o = scal_ref[0, 0, 5]
    m_o = scal_ref[0, 0, 6]
    v_o = scal_ref[0, 0, 7]

    s_in = jnp.sqrt(v_in + 1e-5)
    thrx = (qv - b_in) / w_in * s_in + m_in
    io128 = lax.broadcasted_iota(jnp.int32, (1, 128), 1).astype(jnp.float32)
    thr_ref[0] = jnp.where(io128 <= 98.0, thrx, jnp.float32(jnp.inf))

    s_o = jnp.sqrt(v_o + 1e-5)
    tabv = (qpad - m_o) / s_o * w_o + b_o
    k1 = qpad_ref[0, 0] / s_o * w_o
    k2 = b_o - m_o / s_o * w_o
    tab_ref[0] = jnp.where(io128 <= 99.0, tabv,
                           jnp.where(io128 == 100.0, k1, k2))


def _prep(vc, qpad, scal):
    n_c = vc.shape[0]
    out = jax.ShapeDtypeStruct((n_c, 1, 128), jnp.float32)
    thr, tab = pl.pallas_call(
        _prep_body,
        grid=(n_c,),
        in_specs=[
            pl.BlockSpec((1, 1, _PAD), lambda c: (c, 0, 0)),
            pl.BlockSpec((1, _PAD, 1), lambda c: (c, 0, 0)),
            pl.BlockSpec((1, 128), lambda c: (0, 0)),
            pl.BlockSpec((1, 1, 8), lambda c: (c, 0, 0)),
        ],
        out_specs=[pl.BlockSpec((1, 1, 128), lambda c: (c, 0, 0)),
                   pl.BlockSpec((1, 1, 128), lambda c: (c, 0, 0))],
        out_shape=[out, out],
    )(vc.reshape(n_c, 1, _PAD), vc.reshape(n_c, _PAD, 1), qpad,
      scal.reshape(n_c, 1, 8))
    return thr.reshape(n_c, 128), tab.reshape(n_c, 128)


_CHUNK = 12544
_UNROLL = 8
_SC_SLABS = 192


def _tc_body(x_ref, thr_ref, tab_ref, y_ref):
    v = x_ref[0]
    acc = jnp.zeros_like(v)
    for k in range(99):
        acc = acc + jnp.where(thr_ref[0, 0, k] <= v, 1.0, 0.0)
    k1 = tab_ref[0, 0, 100]
    k2 = tab_ref[0, 0, 101]
    y_ref[0] = (acc + 1.0) * k1 + k2


def _tc_bucketize(x3, thr, tab, first_slab, n_tc, n_c):
    rows, cols = x3.shape[1], x3.shape[2]
    return pl.pallas_call(
        _tc_body,
        grid=(n_tc,),
        in_specs=[
            pl.BlockSpec((1, rows, cols), lambda i: (first_slab + i, 0, 0)),
            pl.BlockSpec((1, 1, 128),
                         lambda i: (lax.rem(first_slab + i, n_c), 0, 0)),
            pl.BlockSpec((1, 1, 128),
                         lambda i: (lax.rem(first_slab + i, n_c), 0, 0)),
        ],
        out_specs=pl.BlockSpec((1, rows, cols), lambda i: (i, 0, 0)),
        out_shape=jax.ShapeDtypeStruct((n_tc, rows, cols), jnp.float32),
    )(x3, thr.reshape(n_c, 1, 128), tab.reshape(n_c, 1, 128))


def _sc_bucketize(xf, thr, tab, n_slab, slab_len, n_c):
    info = plsc.get_sparse_core_info()
    nw = info.num_cores * info.num_subcores
    slabs_per_w = n_slab // nw
    n_chunk = slab_len // _CHUNK
    mesh = plsc.VectorSubcoreMesh(core_axis_name="c", subcore_axis_name="s")

    @functools.partial(
        pl.kernel, mesh=mesh,
        out_type=jax.ShapeDtypeStruct((n_slab * slab_len,), jnp.float32),
        compiler_params=pltpu.CompilerParams(needs_layout_passes=False),
        scratch_types=[
            pltpu.VMEM((128,), jnp.float32),
            pltpu.VMEM((128,), jnp.float32),
            pltpu.VMEM((_CHUNK,), jnp.float32),
            pltpu.VMEM((_CHUNK,), jnp.float32),
            pltpu.VMEM((_CHUNK,), jnp.float32),
            pltpu.VMEM((_CHUNK,), jnp.float32),
            pltpu.SemaphoreType.DMA,
            pltpu.SemaphoreType.DMA,
            pltpu.SemaphoreType.DMA,
            pltpu.SemaphoreType.DMA,
        ],
    )
    def body(x_hbm, thr_hbm, tab_hbm, y_hbm, thr_v, tab_v,
             xin0, xin1, yout0, yout1, sin0, sin1, sout0, sout1):
        wid = lax.axis_index("s") * info.num_cores + lax.axis_index("c")
        xin = (xin0, xin1)
        yout = (yout0, yout1)
        sin = (sin0, sin1)
        sout = (sout0, sout1)

        def slab_loop(t, _):
            sl = wid * slabs_per_w + t
            ch = lax.rem(sl, n_c)
            pltpu.sync_copy(thr_hbm.at[ch], thr_v)
            pltpu.sync_copy(tab_hbm.at[ch], tab_v)
            def bcast(i):
                return plsc.load_gather(thr_v, [jnp.full((16,), i, jnp.int32)])

            t63 = bcast(63)
            t31, t95 = bcast(31), bcast(95)
            t15, t47, t79, t111 = bcast(15), bcast(47), bcast(79), bcast(111)
            tl8 = [bcast(7 + 16 * i) for i in range(8)]
            k1 = plsc.load_gather(tab_v, [jnp.full((16,), 100, jnp.int32)])
            k2 = plsc.load_gather(tab_v, [jnp.full((16,), 101, jnp.int32)])

            def search(src, dst):
                @plsc.parallel_loop(0, _CHUNK, 16, unroll=_UNROLL)
                def _(off):
                    xv = src[pl.ds(off, 16)]
                    c64 = t63 <= xv
                    idx = jnp.where(c64, jnp.int32(64), jnp.int32(0))
                    tm = jnp.where(c64, t95, t31)
                    c32 = tm <= xv
                    idx = jnp.where(c32, idx + 32, idx)
                    tm = jnp.where(c64, jnp.where(c32, t111, t79),
                                   jnp.where(c32, t47, t15))
                    c16 = tm <= xv
                    idx = jnp.where(c16, idx + 16, idx)
                    hi = jnp.where(c64, jnp.where(c32, tl8[7], tl8[5]),
                                   jnp.where(c32, tl8[3], tl8[1]))
                    lo = jnp.where(c64, jnp.where(c32, tl8[6], tl8[4]),
                                   jnp.where(c32, tl8[2], tl8[0]))
                    tm = jnp.where(c16, hi, lo)
                    idx = jnp.where(tm <= xv, idx + 8, idx)
                    for s in (4, 2, 1):
                        t_ = plsc.load_gather(thr_v, [idx + (s - 1)])
                        idx = jnp.where(t_ <= xv, idx + s, idx)
                    ov = (idx + 1).astype(jnp.float32) * k1 + k2
                    dst[pl.ds(off, 16)] = ov

            base0 = sl * slab_len
            hin = [None, None]
            hout = [None, None]
            hin[0] = pltpu.async_copy(x_hbm.at[pl.ds(base0, _CHUNK)],
                                      xin[0], sin[0])
            for k in range(n_chunk):
                b = k & 1
                if k + 1 < n_chunk:
                    hin[1 - b] = pltpu.async_copy(
                        x_hbm.at[pl.ds(base0 + (k + 1) * _CHUNK, _CHUNK)],
                        xin[1 - b], sin[1 - b])
                hin[b].wait()
                if hout[b] is not None:
                    hout[b].wait()
                search(xin[b], yout[b])
                hout[b] = pltpu.async_copy(
                    yout[b], y_hbm.at[pl.ds(base0 + k * _CHUNK, _CHUNK)],
                    sout[b])
            for b in range(2):
                if hout[b] is not None:
                    hout[b].wait()
            return 0

        lax.fori_loop(0, slabs_per_w, slab_loop, 0)

    return body(xf, thr, tab)


def kernel(x, bn_weight, bn_bias, bn_mean, bn_var, bn_out_weight, bn_out_bias,
           bn_out_mean, bn_out_var, quant_list, context):
    b, n_c, h, w = x.shape
    n_ctx = context.shape[1]
    nq = quant_list.shape[0]

    minv = jnp.full((n_c, 1), -100.0, jnp.float32)
    maxv = jnp.full((n_c, 1), 100.0, jnp.float32)
    padv = jnp.full((n_c, _PAD - n_ctx - 2), _BIG, jnp.float32)
    vc = jnp.concatenate([context, minv, maxv, padv], axis=1)
    qpad = jnp.concatenate([quant_list, jnp.zeros((128 - nq,), jnp.float32)])
    qpad = qpad.reshape(1, 128)
    scal = jnp.stack([bn_weight, bn_bias, bn_mean, bn_var,
                      bn_out_weight, bn_out_bias, bn_out_mean, bn_out_var],
                     axis=1)

    thr, tab = _prep(vc, qpad, scal)

    slab_len = h * w
    n_slab = b * n_c
    xf = x.reshape(-1)
    y_sc = _sc_bucketize(xf, thr, tab, _SC_SLABS, slab_len, n_c)
    x3 = x.reshape(n_slab, slab_len // 128, 128)
    y_tc = _tc_bucketize(x3, thr, tab, _SC_SLABS, n_slab - _SC_SLABS, n_c)
    y = jnp.concatenate(
        [y_sc.reshape(_SC_SLABS, slab_len),
         y_tc.reshape(n_slab - _SC_SLABS, slab_len)], axis=0)
    return y.reshape(x.shape)


# speedup vs baseline: 1.1803x; 1.1803x over previous
import functools

import jax
import jax.numpy as jnp
from jax import lax
from jax.experimental import pallas as pl
from jax.experimental.pallas import tpu as pltpu
from jax.experimental.pallas import tpu_sc as plsc

_PAD = 1024
_NREAL = 1002
_NSAMP = 1000.0
_BIG = 1e30


def _prep_body(vc_ref, vcT_ref, qpad_ref, scal_ref, thr_ref, tab_ref):
    vrow = vc_ref[0]
    vcol = vcT_ref[0]
    qpad = qpad_ref[...]

    io_row = lax.broadcasted_iota(jnp.int32, (1, _PAD), 1).astype(jnp.float32)
    io_col = lax.broadcasted_iota(jnp.int32, (_PAD, 1), 0).astype(jnp.float32)

    before = (vrow < vcol) | ((vrow == vcol) & (io_row < io_col))
    before_f = jnp.where(before, 1.0, 0.0)
    ones_col = jnp.ones((_PAD, 1), jnp.float32)
    rank_col = jax.lax.dot_general(
        before_f, ones_col, (((1,), (0,)), ((), ())),
        preferred_element_type=jnp.float32)

    sn = jnp.sum(jnp.where(vrow < 0, 1.0, 0.0))
    sp = jnp.float32(_NREAL) - sn
    wn = (1.0 / sn) * _NSAMP
    wp = (1.0 / sp) * _NSAMP
    tot = sn * wn + sp * wp

    cw_col = jnp.where(io_col < sn, (io_col + 1.0) * wn,
                       sn * wn + (io_col + 1.0 - sn) * wp)
    tw = qpad * tot

    valid = io_col < jnp.float32(_NREAL)
    above = jnp.sum(jnp.where((cw_col <= tw) & valid, 1.0, 0.0),
                    axis=0, keepdims=True)
    below = jnp.clip(above - 1.0, 0.0, _NSAMP - 1.0)
    above = jnp.clip(above, 0.0, _NSAMP - 1.0)

    def sorted_at(r):
        return jnp.sum(jnp.where(rank_col == r, vcol, 0.0), axis=0, keepdims=True)

    def cw_at(r):
        return jnp.where(r < sn, (r + 1.0) * wn, sn * wn + (r + 1.0 - sn) * wp)

    vb, va = sorted_at(below), sorted_at(above)
    wb, wa = cw_at(below), cw_at(above)
    frac = (tw - wb) / (wa - wb + 1e-6)
    qv = vb + (va - vb) * frac

    w_in = scal_ref[0, 0, 0]
    b_in = scal_ref[0, 0, 1]
    m_in = scal_ref[0, 0, 2]
    v_in = scal_ref[0, 0, 3]
    w_o = scal_ref[0, 0, 4]
    b_o = scal_ref[0, 0, 5]
    m_o = scal_ref[0, 0, 6]
    v_o = scal_ref[0, 0, 7]

    s_in = jnp.sqrt(v_in + 1e-5)
    thrx = (qv - b_in) / w_in * s_in + m_in
    io128 = lax.broadcasted_iota(jnp.int32, (1, 128), 1).astype(jnp.float32)
    thr_ref[0] = jnp.where(io128 <= 98.0, thrx, jnp.float32(jnp.inf))

    s_o = jnp.sqrt(v_o + 1e-5)
    tabv = (qpad - m_o) / s_o * w_o + b_o
    k1 = qpad_ref[0, 0] / s_o * w_o
    k2 = b_o - m_o / s_o * w_o
    tab_ref[0] = jnp.where(io128 <= 99.0, tabv,
                           jnp.where(io128 == 100.0, k1, k2))


def _prep(vc, qpad, scal):
    n_c = vc.shape[0]
    out = jax.ShapeDtypeStruct((n_c, 1, 128), jnp.float32)
    thr, tab = pl.pallas_call(
        _prep_body,
        grid=(n_c,),
        in_specs=[
            pl.BlockSpec((1, 1, _PAD), lambda c: (c, 0, 0)),
            pl.BlockSpec((1, _PAD, 1), lambda c: (c, 0, 0)),
            pl.BlockSpec((1, 128), lambda c: (0, 0)),
            pl.BlockSpec((1, 1, 8), lambda c: (c, 0, 0)),
        ],
        out_specs=[pl.BlockSpec((1, 1, 128), lambda c: (c, 0, 0)),
                   pl.BlockSpec((1, 1, 128), lambda c: (c, 0, 0))],
        out_shape=[out, out],
    )(vc.reshape(n_c, 1, _PAD), vc.reshape(n_c, _PAD, 1), qpad,
      scal.reshape(n_c, 1, 8))
    return thr.reshape(n_c, 128), tab.reshape(n_c, 128)


_CHUNK = 12544
_UNROLL = 8
_SC_SLABS = 192


def _tc_body(x_ref, thr_ref, tab_ref, y_ref):
    v = x_ref[0]
    acc = jnp.zeros_like(v)
    for k in range(99):
        acc = acc + jnp.where(thr_ref[0, 0, k] <= v, 1.0, 0.0)
    k1 = tab_ref[0, 0, 100]
    k2 = tab_ref[0, 0, 101]
    y_ref[0] = (acc + 1.0) * k1 + k2


def _tc_bucketize(x3, thr, tab, first_slab, n_tc, n_c):
    rows, cols = x3.shape[1], x3.shape[2]
    return pl.pallas_call(
        _tc_body,
        grid=(n_tc,),
        in_specs=[
            pl.BlockSpec((1, rows, cols), lambda i: (first_slab + i, 0, 0)),
            pl.BlockSpec((1, 1, 128),
                         lambda i: (lax.rem(first_slab + i, n_c), 0, 0)),
            pl.BlockSpec((1, 1, 128),
                         lambda i: (lax.rem(first_slab + i, n_c), 0, 0)),
        ],
        out_specs=pl.BlockSpec((1, rows, cols), lambda i: (i, 0, 0)),
        out_shape=jax.ShapeDtypeStruct((n_tc, rows, cols), jnp.float32),
    )(x3, thr.reshape(n_c, 1, 128), tab.reshape(n_c, 1, 128))


def _sc_bucketize(xf, thr, tab, n_slab, slab_len, n_c):
    info = plsc.get_sparse_core_info()
    nw = info.num_cores * info.num_subcores
    slabs_per_w = n_slab // nw
    n_chunk = slab_len // _CHUNK
    mesh = plsc.VectorSubcoreMesh(core_axis_name="c", subcore_axis_name="s")

    @functools.partial(
        pl.kernel, mesh=mesh,
        out_type=jax.ShapeDtypeStruct((n_slab * slab_len,), jnp.float32),
        compiler_params=pltpu.CompilerParams(needs_layout_passes=False),
        scratch_types=[
            pltpu.VMEM((128,), jnp.float32),
            pltpu.VMEM((128,), jnp.float32),
            pltpu.VMEM((_CHUNK,), jnp.float32),
            pltpu.VMEM((_CHUNK,), jnp.float32),
            pltpu.VMEM((_CHUNK,), jnp.float32),
            pltpu.VMEM((_CHUNK,), jnp.float32),
            pltpu.SemaphoreType.DMA,
            pltpu.SemaphoreType.DMA,
            pltpu.SemaphoreType.DMA,
            pltpu.SemaphoreType.DMA,
        ],
    )
    def body(x_hbm, thr_hbm, tab_hbm, y_hbm, thr_v, tab_v,
             xin0, xin1, yout0, yout1, sin0, sin1, sout0, sout1):
        wid = lax.axis_index("s") * info.num_cores + lax.axis_index("c")
        xin = (xin0, xin1)
        yout = (yout0, yout1)
        sin = (sin0, sin1)
        sout = (sout0, sout1)

        def slab_loop(t, _):
            sl = wid * slabs_per_w + t
            ch = lax.rem(sl, n_c)
            pltpu.sync_copy(thr_hbm.at[ch], thr_v)
            pltpu.sync_copy(tab_hbm.at[ch], tab_v)
            def bcast(i):
                return plsc.load_gather(thr_v, [jnp.full((16,), i, jnp.int32)])

            t63 = bcast(63)
            t31, t95 = bcast(31), bcast(95)
            t15, t47, t79, t111 = bcast(15), bcast(47), bcast(79), bcast(111)
            tl8 = [bcast(7 + 16 * i) for i in range(8)]
            k1 = plsc.load_gather(tab_v, [jnp.full((16,), 100, jnp.int32)])
            k2 = plsc.load_gather(tab_v, [jnp.full((16,), 101, jnp.int32)])

            def search(src, dst):
                @plsc.parallel_loop(0, _CHUNK, 16, unroll=_UNROLL)
                def _(off):
                    xv = src[pl.ds(off, 16)]
                    c64 = t63 <= xv
                    idx = jnp.where(c64, jnp.int32(64), jnp.int32(0))
                    tm = jnp.where(c64, t95, t31)
                    c32 = tm <= xv
                    idx = jnp.where(c32, idx + 32, idx)
                    tm = jnp.where(c64, jnp.where(c32, t111, t79),
                                   jnp.where(c32, t47, t15))
                    c16 = tm <= xv
                    idx = jnp.where(c16, idx + 16, idx)
                    hi = jnp.where(c64, jnp.where(c32, tl8[7], tl8[5]),
                                   jnp.where(c32, tl8[3], tl8[1]))
                    lo = jnp.where(c64, jnp.where(c32, tl8[6], tl8[4]),
                                   jnp.where(c32, tl8[2], tl8[0]))
                    tm = jnp.where(c16, hi, lo)
                    idx = jnp.where(tm <= xv, idx + 8, idx)
                    for s in (4, 2, 1):
                        t_ = plsc.load_gather(thr_v, [idx + (s - 1)])
                        idx = jnp.where(t_ <= xv, idx + s, idx)
                    ov = (idx + 1).astype(jnp.float32) * k1 + k2
                    dst[pl.ds(off, 16)] = ov

            base0 = sl * slab_len
            hin = [None, None]
            hout = [None, None]
            hin[0] = pltpu.async_copy(x_hbm.at[pl.ds(base0, _CHUNK)],
                                      xin[0], sin[0])
            for k in range(n_chunk):
                b = k & 1
                if k + 1 < n_chunk:
                    hin[1 - b] = pltpu.async_copy(
                        x_hbm.at[pl.ds(base0 + (k + 1) * _CHUNK, _CHUNK)],
                        xin[1 - b], sin[1 - b])
                hin[b].wait()
                if hout[b] is not None:
                    hout[b].wait()
                search(xin[b], yout[b])
                hout[b] = pltpu.async_copy(
                    yout[b], y_hbm.at[pl.ds(base0 + k * _CHUNK, _CHUNK)],
                    sout[b])
            for b in range(2):
                if hout[b] is not None:
                    hout[b].wait()
            return 0

        lax.fori_loop(0, slabs_per_w, slab_loop, 0)

    return body(xf, thr, tab)


def kernel(x, bn_weight, bn_bias, bn_mean, bn_var, bn_out_weight, bn_out_bias,
           bn_out_mean, bn_out_var, quant_list, context):
    b, n_c, h, w = x.shape
    n_ctx = context.shape[1]
    nq = quant_list.shape[0]

    minv = jnp.full((n_c, 1), -100.0, jnp.float32)
    maxv = jnp.full((n_c, 1), 100.0, jnp.float32)
    padv = jnp.full((n_c, _PAD - n_ctx - 2), _BIG, jnp.float32)
    vc = jnp.concatenate([context, minv, maxv, padv], axis=1)
    qpad = jnp.concatenate([quant_list, jnp.zeros((128 - nq,), jnp.float32)])
    qpad = qpad.reshape(1, 128)
    scal = jnp.stack([bn_weight, bn_bias, bn_mean, bn_var,
                      bn_out_weight, bn_out_bias, bn_out_mean, bn_out_var],
                     axis=1)

    thr, tab = _prep(vc, qpad, scal)

    slab_len = h * w
    n_slab = b * n_c
    xf = x.reshape(-1)
    y = _sc_bucketize(xf, thr, tab, n_slab, slab_len, n_c)
    return y.reshape(x.shape)
